# baseline (speedup 1.0000x reference)
import jax
import jax.numpy as jnp
from jax.experimental import pallas as pl

_B = 2
_N = 40000
_DIM = 768
_G = 256
_M = 64


def _fps(xyz, n_samples):
    Bb, Nn, _ = xyz.shape
    centroids = jnp.zeros((Bb, n_samples), dtype=jnp.int32)
    distance = jnp.full((Bb, Nn), 1e10, dtype=xyz.dtype)
    farthest = jnp.zeros((Bb,), dtype=jnp.int32)

    def body(i, carry):
        centroids, distance, farthest = carry
        centroids = centroids.at[:, i].set(farthest)
        centroid = xyz[jnp.arange(Bb), farthest][:, None, :]
        dist = jnp.sum((xyz - centroid) ** 2, axis=-1)
        distance = jnp.minimum(distance, dist)
        farthest = jnp.argmax(distance, axis=-1).astype(jnp.int32)
        return centroids, distance, farthest

    centroids, _, _ = jax.lax.fori_loop(0, n_samples, body,
                                        (centroids, distance, farthest))
    return centroids


def _noop_pallas(x):
    def body(x_ref, o_ref):
        o_ref[...] = x_ref[...]
    return pl.pallas_call(
        body, out_shape=jax.ShapeDtypeStruct(x.shape, x.dtype))(x)


def kernel(xyzs, pointcloud_features, level):
    Bb, _, dim = pointcloud_features.shape
    xyz = xyzs[:, :_N, :]
    scene_fts_in = pointcloud_features[:, :_N, :]
    points = jnp.concatenate([xyz, scene_fts_in], axis=-1)

    pxyz = points[..., :3]
    fps_idx = _fps(jax.lax.stop_gradient(pxyz), _G)
    bidx = jnp.arange(Bb)[:, None]
    center = pxyz[bidx, fps_idx]
    d2 = (jnp.sum(center ** 2, axis=-1)[:, :, None]
          - 2.0 * jnp.einsum('bgc,bnc->bgn', center, pxyz)
          + jnp.sum(pxyz ** 2, axis=-1)[:, None, :])
    _, idx = jax.lax.top_k(-jax.lax.stop_gradient(d2), _M)
    bidx2 = jnp.arange(Bb)[:, None, None]
    neighborhood = points[bidx2, idx]
    nxyz = neighborhood[..., :3] - center[:, :, None, :]

    scene_fts = neighborhood[..., 3:].mean(-2)
    all_fts = _noop_pallas(scene_fts)
    all_fts_mask = jnp.ones((Bb, _G), dtype=pointcloud_features.dtype)
    return all_fts, all_fts_mask, center, nxyz


# speedup vs baseline: 1.4505x; 1.4505x over previous
import functools

import jax
import jax.numpy as jnp
from jax import lax
from jax.experimental import pallas as pl
from jax.experimental.pallas import tpu as pltpu

_B = 2
_N = 40000
_DIM = 768
_G = 256
_M = 64
_NPAD = 40960
_ROWS = _NPAD // 128


def _fps_body(x_ref, cent_ref, dist_ref):
    flat = (lax.broadcasted_iota(jnp.int32, (_ROWS, 128), 0) * 128
            + lax.broadcasted_iota(jnp.int32, (_ROWS, 128), 1))
    valid = flat < _N
    lane_iota = lax.broadcasted_iota(jnp.int32, (1, 128), 1)
    for b in range(_B):
        dist_ref[b] = jnp.where(valid, jnp.float32(1e10), jnp.float32(-1.0))

    def step(i, fars):
        new_fars = []
        for b in range(_B):
            far = fars[b]
            row = far // 128
            lane = far - row * 128
            xr = x_ref[b, 0, pl.ds(row, 1), :]
            yr = x_ref[b, 1, pl.ds(row, 1), :]
            zr = x_ref[b, 2, pl.ds(row, 1), :]
            sel = lane_iota == lane
            cx = jnp.sum(jnp.where(sel, xr, 0.0))
            cy = jnp.sum(jnp.where(sel, yr, 0.0))
            cz = jnp.sum(jnp.where(sel, zr, 0.0))
            c2 = cx * cx + cy * cy + cz * cz
            rowvec = jnp.where(
                lane_iota == 0, cx,
                jnp.where(lane_iota == 1, cy,
                          jnp.where(lane_iota == 2, cz,
                                    jnp.where(lane_iota == 3, c2, 0.0))))
            cent_ref[pl.ds(b * _G + i, 1), :] = rowvec
            dx = x_ref[b, 0] - cx
            dy = x_ref[b, 1] - cy
            dz = x_ref[b, 2] - cz
            d = (dx * dx + dy * dy) + dz * dz
            nd = jnp.minimum(dist_ref[b], d)
            dist_ref[b] = nd
            maxv = jnp.max(jnp.max(nd, axis=0, keepdims=True))
            newfar = jnp.min(jnp.where(nd == maxv, flat, jnp.int32(2**30)))
            new_fars.append(newfar)
        return tuple(new_fars)

    lax.fori_loop(0, _G, step, tuple(jnp.int32(0) for _ in range(_B)))


def _run_fps(xpad):
    return pl.pallas_call(
        _fps_body,
        out_shape=jax.ShapeDtypeStruct((_B * _G, 128), jnp.float32),
        scratch_shapes=[pltpu.VMEM((_B, _ROWS, 128), jnp.float32)],
    )(xpad)


def kernel(xyzs, pointcloud_features, level):
    Bb = _B
    xyz = xyzs[:, :_N, :]
    feats = pointcloud_features[:, :_N, :]

    xt = jnp.transpose(xyz, (0, 2, 1))
    xpad = jnp.pad(xt, ((0, 0), (0, 0), (0, _NPAD - _N)))
    xpad = xpad.reshape(_B, 3, _ROWS, 128)
    cent = _run_fps(xpad)
    center = cent[:, :3].reshape(_B, _G, 3)

    d2 = (jnp.sum(center ** 2, axis=-1)[:, :, None]
          - 2.0 * jnp.einsum('bgc,bnc->bgn', center, xyz)
          + jnp.sum(xyz ** 2, axis=-1)[:, None, :])
    _, idx = lax.top_k(-lax.stop_gradient(d2), _M)
    bidx2 = jnp.arange(Bb)[:, None, None]
    nxyz = xyz[bidx2, idx] - center[:, :, None, :]
    scene_fts = feats[bidx2, idx].mean(-2)

    all_fts_mask = jnp.ones((Bb, _G), dtype=pointcloud_features.dtype)
    return scene_fts, all_fts_mask, center, nxyz
